# speedup vs baseline: 6.8226x; 6.8226x over previous
import functools

import jax
import jax.numpy as jnp
from jax import lax
from jax.experimental import pallas as pl
from jax.experimental.pallas import tpu as pltpu
from jax.experimental.pallas import tpu_sc as plsc

NUM_TOKENS = 4096 * 200
DIM = 128
NUM_WORKERS = 32
ROWS_PER_WORKER = NUM_TOKENS // NUM_WORKERS
GATHER = 128
CHUNK = 256
NGATHER = CHUNK // GATHER
NCHUNKS = ROWS_PER_WORKER // CHUNK

_mesh = plsc.VectorSubcoreMesh(core_axis_name="c", subcore_axis_name="s")


@functools.partial(
    pl.kernel,
    mesh=_mesh,
    out_type=jax.ShapeDtypeStruct((NUM_TOKENS, DIM), jnp.float32),
    scratch_types=[
        pltpu.VMEM((NGATHER, GATHER), jnp.int32),
        pltpu.VMEM((CHUNK, DIM), jnp.float32),
        pltpu.SemaphoreType.DMA,
    ],
)
def _embed(ids_hbm, table_hbm, out_hbm, idx_v, rows_v, sem):
    wid = lax.axis_index("s") * 2 + lax.axis_index("c")
    row_base = wid * (ROWS_PER_WORKER // GATHER)

    def body(ci, _):
        idx_row = row_base + ci * NGATHER
        tok_off = idx_row * GATHER
        pltpu.sync_copy(ids_hbm.at[pl.ds(idx_row, NGATHER)], idx_v)
        for j in range(NGATHER):
            pltpu.async_copy(
                table_hbm.at[idx_v.at[j]],
                rows_v.at[pl.ds(j * GATHER, GATHER)],
                sem,
            )
        for j in range(NGATHER):
            pltpu.make_async_copy(
                table_hbm.at[idx_v.at[j]],
                rows_v.at[pl.ds(j * GATHER, GATHER)],
                sem,
            ).wait()
        pltpu.sync_copy(rows_v, out_hbm.at[pl.ds(tok_off, CHUNK)])
        return 0

    lax.fori_loop(0, NCHUNKS, body, 0)


def kernel(token_ids, weight):
    ids = token_ids.astype(jnp.int32).reshape(NUM_TOKENS // GATHER, GATHER)
    out = _embed(ids, weight)
    return out.reshape(token_ids.shape + (DIM,))


# speedup vs baseline: 9.1624x; 1.3430x over previous
import functools

import jax
import jax.numpy as jnp
from jax import lax
from jax.experimental import pallas as pl
from jax.experimental.pallas import tpu as pltpu
from jax.experimental.pallas import tpu_sc as plsc

NUM_TOKENS = 4096 * 200
DIM = 128
NUM_WORKERS = 32
ROWS_PER_WORKER = NUM_TOKENS // NUM_WORKERS
CHUNK = 128
NCHUNKS = ROWS_PER_WORKER // CHUNK
NBUF = 4
NGROUPS = NCHUNKS // NBUF

_mesh = plsc.VectorSubcoreMesh(core_axis_name="c", subcore_axis_name="s")


@functools.partial(
    pl.kernel,
    mesh=_mesh,
    out_type=jax.ShapeDtypeStruct((NUM_TOKENS, DIM), jnp.float32),
    scratch_types=[
        pltpu.VMEM((NCHUNKS, CHUNK), jnp.int32),
        pltpu.VMEM((NBUF, CHUNK, DIM), jnp.float32),
        pltpu.SemaphoreType.DMA,
        pltpu.SemaphoreType.DMA,
        pltpu.SemaphoreType.DMA,
        pltpu.SemaphoreType.DMA,
        pltpu.SemaphoreType.DMA,
        pltpu.SemaphoreType.DMA,
        pltpu.SemaphoreType.DMA,
        pltpu.SemaphoreType.DMA,
    ],
)
def _embed(ids_hbm, table_hbm, out_hbm, idx_v, rows_v,
           g0, g1, g2, g3, s0, s1, s2, s3):
    gsem = (g0, g1, g2, g3)
    ssem = (s0, s1, s2, s3)
    wid = lax.axis_index("s") * 2 + lax.axis_index("c")
    chunk_base = wid * NCHUNKS
    tok_base = chunk_base * CHUNK

    pltpu.sync_copy(ids_hbm.at[pl.ds(chunk_base, NCHUNKS)], idx_v)

    def fire_gather(ci, slot):
        pltpu.async_copy(table_hbm.at[idx_v.at[ci]], rows_v.at[slot], gsem[slot])

    def wait_gather(ci, slot):
        pltpu.make_async_copy(
            table_hbm.at[idx_v.at[ci]], rows_v.at[slot], gsem[slot]).wait()

    def fire_store(ci, slot):
        pltpu.async_copy(
            rows_v.at[slot],
            out_hbm.at[pl.ds(tok_base + ci * CHUNK, CHUNK)],
            ssem[slot])

    def wait_store(ci, slot):
        pltpu.make_async_copy(
            rows_v.at[slot],
            out_hbm.at[pl.ds(tok_base + ci * CHUNK, CHUNK)],
            ssem[slot]).wait()

    for b in range(NBUF - 1):
        fire_gather(b, b)

    def group(g, _):
        for b in range(NBUF):
            ci = g * NBUF + b
            nslot = (b + NBUF - 1) % NBUF
            nci = ci + NBUF - 1

            @pl.when(jnp.logical_and(nci < NCHUNKS, ci >= 1))
            def _():
                wait_store(ci - 1, nslot)

            @pl.when(nci < NCHUNKS)
            def _():
                fire_gather(nci, nslot)

            wait_gather(ci, b)
            fire_store(ci, b)
        return 0

    lax.fori_loop(0, NGROUPS, group, 0)

    for b in range(NBUF):
        ci = NCHUNKS - NBUF + b
        wait_store(ci, b)


def kernel(token_ids, weight):
    ids = token_ids.astype(jnp.int32).reshape(NUM_TOKENS // CHUNK, CHUNK)
    out = _embed(ids, weight)
    return out.reshape(token_ids.shape + (DIM,))


# speedup vs baseline: 9.1784x; 1.0017x over previous
import functools

import jax
import jax.numpy as jnp
from jax import lax
from jax.experimental import pallas as pl
from jax.experimental.pallas import tpu as pltpu
from jax.experimental.pallas import tpu_sc as plsc

NUM_TOKENS = 4096 * 200
DIM = 128
NUM_WORKERS = 32
ROWS_PER_WORKER = NUM_TOKENS // NUM_WORKERS
CHUNK = 128
NCHUNKS = ROWS_PER_WORKER // CHUNK
NBUF = 5
NGROUPS = NCHUNKS // NBUF

_mesh = plsc.VectorSubcoreMesh(core_axis_name="c", subcore_axis_name="s")


@functools.partial(
    pl.kernel,
    mesh=_mesh,
    out_type=jax.ShapeDtypeStruct((NUM_TOKENS, DIM), jnp.float32),
    scratch_types=[
        pltpu.VMEM((NCHUNKS, CHUNK), jnp.int32),
        pltpu.VMEM((NBUF, CHUNK, DIM), jnp.float32),
    ] + [pltpu.SemaphoreType.DMA] * (2 * NBUF),
)
def _embed(ids_hbm, table_hbm, out_hbm, idx_v, rows_v, *sems):
    gsem = sems[:NBUF]
    ssem = sems[NBUF:]
    wid = lax.axis_index("s") * 2 + lax.axis_index("c")
    chunk_base = wid * NCHUNKS
    tok_base = chunk_base * CHUNK

    pltpu.sync_copy(ids_hbm.at[pl.ds(chunk_base, NCHUNKS)], idx_v)

    def fire_gather(ci, slot):
        pltpu.async_copy(table_hbm.at[idx_v.at[ci]], rows_v.at[slot], gsem[slot])

    def wait_gather(ci, slot):
        pltpu.make_async_copy(
            table_hbm.at[idx_v.at[ci]], rows_v.at[slot], gsem[slot]).wait()

    def fire_store(ci, slot):
        pltpu.async_copy(
            rows_v.at[slot],
            out_hbm.at[pl.ds(tok_base + ci * CHUNK, CHUNK)],
            ssem[slot])

    def wait_store(ci, slot):
        pltpu.make_async_copy(
            rows_v.at[slot],
            out_hbm.at[pl.ds(tok_base + ci * CHUNK, CHUNK)],
            ssem[slot]).wait()

    for b in range(NBUF - 1):
        fire_gather(b, b)

    def group(g, _):
        for b in range(NBUF):
            ci = g * NBUF + b
            nslot = (b + NBUF - 1) % NBUF
            nci = ci + NBUF - 1

            @pl.when(jnp.logical_and(nci < NCHUNKS, ci >= 1))
            def _():
                wait_store(ci - 1, nslot)

            @pl.when(nci < NCHUNKS)
            def _():
                fire_gather(nci, nslot)

            wait_gather(ci, b)
            fire_store(ci, b)
        return 0

    lax.fori_loop(0, NGROUPS, group, 0)

    for b in range(NBUF):
        ci = NCHUNKS - NBUF + b
        wait_store(ci, b)


def kernel(token_ids, weight):
    ids = token_ids.astype(jnp.int32).reshape(NUM_TOKENS // CHUNK, CHUNK)
    out = _embed(ids, weight)
    return out.reshape(token_ids.shape + (DIM,))
